# baseline (speedup 1.0000x reference)
import jax
import jax.numpy as jnp
from jax import lax
from jax.experimental import pallas as pl
from jax.experimental.pallas import tpu as pltpu

_NUM_PROPS = 9
_NUM_RULES = 8
_NUM_PREMISES = 2
_OUT_DIM = 27
_RP = _NUM_RULES * _NUM_PREMISES
_SIM_ROWS = _NUM_PROPS * _RP
_K36 = _NUM_PROPS * 4
_BB = 32768


def _block_kernel(*refs):
    bs_refs = refs[:_NUM_PROPS]
    p0_ref, p1_ref, headst_ref, bias_ref, out_ref = refs[_NUM_PROPS:]
    bb = bs_refs[0].shape[1]

    c_iota = lax.broadcasted_iota(jnp.int32, (_SIM_ROWS, 3), 1)
    n_iota = lax.broadcasted_iota(jnp.int32, (_SIM_ROWS, 3), 0)
    player = jnp.where(c_iota == 1, 1.0, jnp.where(c_iota == 2, -1.0, 0.0))
    pos = ((n_iota // _RP).astype(jnp.float32) - 4.0) * 0.25
    d0 = player - p0_ref[...]
    d1 = pos - p1_ref[...]
    w_all = jnp.exp(-(d0 * d0 + d1 * d1)).astype(jnp.bfloat16)

    sat = None
    for i in range(_NUM_PROPS):
        s_i = lax.dot_general(w_all[i * _RP:(i + 1) * _RP, :],
                              bs_refs[i][...].astype(jnp.bfloat16),
                              (((1,), (0,)), ((), ())),
                              preferred_element_type=jnp.float32)
        sat = s_i if sat is None else jnp.maximum(sat, s_i)

    act = sat[0:_NUM_RULES, :] * sat[_NUM_RULES:_RP, :]

    out36 = lax.dot_general(headst_ref[...], act, (((1,), (0,)), ((), ())),
                            preferred_element_type=jnp.float32)
    out36 = out36 + bias_ref[...]
    out_ref[...] = out36.reshape(_NUM_PROPS, 4, bb)[:, 0:3, :]


def kernel(board_state, premises, heads, bias):
    b = board_state.shape[0]
    bs_views = [board_state[:, i, :].transpose(1, 0) for i in range(_NUM_PROPS)]
    prem_pr = premises.transpose(1, 0, 2).reshape(_RP, _NUM_PREMISES)
    p0b = jnp.broadcast_to(jnp.tile(prem_pr[:, 0], _NUM_PROPS)[:, None],
                           (_SIM_ROWS, 3))
    p1b = jnp.broadcast_to(jnp.tile(prem_pr[:, 1], _NUM_PROPS)[:, None],
                           (_SIM_ROWS, 3))
    heads_t4 = jnp.pad(heads.T.reshape(_NUM_PROPS, 3, _NUM_RULES),
                       ((0, 0), (0, 1), (0, 0))).reshape(_K36, _NUM_RULES)
    bias4 = jnp.pad(bias.reshape(_NUM_PROPS, 3),
                    ((0, 0), (0, 1))).reshape(_K36, 1)

    grid = (b // _BB,)
    out_t = pl.pallas_call(
        _block_kernel,
        grid=grid,
        in_specs=(
            [pl.BlockSpec((3, _BB), lambda i: (0, i))] * _NUM_PROPS
            + [
                pl.BlockSpec((_SIM_ROWS, 3), lambda i: (0, 0)),
                pl.BlockSpec((_SIM_ROWS, 3), lambda i: (0, 0)),
                pl.BlockSpec((_K36, _NUM_RULES), lambda i: (0, 0)),
                pl.BlockSpec((_K36, 1), lambda i: (0, 0)),
            ]
        ),
        out_specs=pl.BlockSpec((_NUM_PROPS, 3, _BB), lambda i: (0, 0, i)),
        out_shape=jax.ShapeDtypeStruct((_NUM_PROPS, 3, b), jnp.float32),
        compiler_params=pltpu.CompilerParams(
            dimension_semantics=("parallel",),
        ),
    )(*bs_views, p0b, p1b, heads_t4, bias4)
    return out_t.transpose(2, 0, 1)


# speedup vs baseline: 1.7031x; 1.7031x over previous
import jax
import jax.numpy as jnp
from jax import lax
from jax.experimental import pallas as pl
from jax.experimental.pallas import tpu as pltpu

_NUM_PROPS = 9
_NUM_RULES = 8
_NUM_PREMISES = 2
_OUT_DIM = 27
_RP = _NUM_RULES * _NUM_PREMISES
_SIM_ROWS = _NUM_PROPS * _RP
_K36 = _NUM_PROPS * 4
_BB = 32768


def _block_kernel(bs_ref, p0_ref, p1_ref, headst_ref, bias_ref, out_ref):
    bb = bs_ref.shape[2]
    k_iota = lax.broadcasted_iota(jnp.int32, (_SIM_ROWS, _K36), 1)
    n_iota = lax.broadcasted_iota(jnp.int32, (_SIM_ROWS, _K36), 0)
    c = k_iota % 4
    i_k = k_iota // 4
    i_n = n_iota // _RP
    player = jnp.where(c == 1, 1.0, jnp.where(c == 2, -1.0, 0.0))
    pos = (i_k.astype(jnp.float32) - 4.0) * 0.25
    d0 = player - p0_ref[...]
    d1 = pos - p1_ref[...]
    w = jnp.exp(-(d0 * d0 + d1 * d1))
    w = jnp.where((i_n == i_k) & (c < 3), w, 0.0)

    bs36 = jnp.pad(bs_ref[...], ((0, 0), (0, 1), (0, 0))).reshape(_K36, bb)
    sim_t = lax.dot_general(w.astype(jnp.bfloat16),
                            bs36.astype(jnp.bfloat16),
                            (((1,), (0,)), ((), ())),
                            preferred_element_type=jnp.float32)

    sat = sim_t[0:_RP, :]
    for i in range(1, _NUM_PROPS):
        sat = jnp.maximum(sat, sim_t[i * _RP:(i + 1) * _RP, :])

    act = sat[0:_NUM_RULES, :] * sat[_NUM_RULES:_RP, :]

    out36 = lax.dot_general(headst_ref[...], act, (((1,), (0,)), ((), ())),
                            preferred_element_type=jnp.float32)
    out36 = out36 + bias_ref[...]
    out_ref[...] = out36.reshape(_NUM_PROPS, 4, bb)[:, 0:3, :]


def kernel(board_state, premises, heads, bias):
    b = board_state.shape[0]
    bs_t = board_state.transpose(1, 2, 0)
    prem_pr = premises.transpose(1, 0, 2).reshape(_RP, _NUM_PREMISES)
    p0b = jnp.broadcast_to(jnp.tile(prem_pr[:, 0], _NUM_PROPS)[:, None],
                           (_SIM_ROWS, _K36))
    p1b = jnp.broadcast_to(jnp.tile(prem_pr[:, 1], _NUM_PROPS)[:, None],
                           (_SIM_ROWS, _K36))
    heads_t4 = jnp.pad(heads.T.reshape(_NUM_PROPS, 3, _NUM_RULES),
                       ((0, 0), (0, 1), (0, 0))).reshape(_K36, _NUM_RULES)
    bias4 = jnp.pad(bias.reshape(_NUM_PROPS, 3),
                    ((0, 0), (0, 1))).reshape(_K36, 1)

    grid = (b // _BB,)
    out_t = pl.pallas_call(
        _block_kernel,
        grid=grid,
        in_specs=[
            pl.BlockSpec((_NUM_PROPS, 3, _BB), lambda i: (0, 0, i)),
            pl.BlockSpec((_SIM_ROWS, _K36), lambda i: (0, 0)),
            pl.BlockSpec((_SIM_ROWS, _K36), lambda i: (0, 0)),
            pl.BlockSpec((_K36, _NUM_RULES), lambda i: (0, 0)),
            pl.BlockSpec((_K36, 1), lambda i: (0, 0)),
        ],
        out_specs=pl.BlockSpec((_NUM_PROPS, 3, _BB), lambda i: (0, 0, i)),
        out_shape=jax.ShapeDtypeStruct((_NUM_PROPS, 3, b), jnp.float32),
        compiler_params=pltpu.CompilerParams(
            dimension_semantics=("parallel",),
        ),
    )(bs_t, p0b, p1b, heads_t4, bias4)
    return out_t.transpose(2, 0, 1)
